# baseline (speedup 1.0000x reference)
import functools

import jax
import jax.numpy as jnp
from jax import lax
from jax.experimental import pallas as pl
from jax.experimental.pallas import tpu as pltpu
from jax.experimental.pallas import tpu_sc as plsc

H = 1024
E = 8
F = 2048
T = 8192
T2 = 2 * T
BLK = 256
NBLK = T2 // BLK + E
NPAD = NBLK * BLK
BE_PAD = 128

NC, NS = 2, 16
NW = NC * NS
EPW = T2 // NW
CH = 32
NCH = EPW // CH


def _router_body(x_ref, wr_ref, gates_ref):
    x = x_ref[...]
    logits = lax.dot_general(x, wr_ref[...], (((1,), (1,)), ((), ())),
                             preferred_element_type=jnp.float32)
    lanes = lax.broadcasted_iota(jnp.int32, logits.shape, 1)
    m0 = jnp.max(logits, axis=1, keepdims=True)
    a0 = jnp.argmax(logits, axis=1).reshape(-1, 1)
    masked = jnp.where(lanes == a0, -jnp.inf, logits)
    m1 = jnp.max(masked, axis=1, keepdims=True)
    a1 = jnp.argmax(masked, axis=1).reshape(-1, 1)
    e1 = jnp.exp(m1 - m0)
    w0 = 1.0 / (1.0 + e1)
    w1 = e1 * w0
    gates_ref[...] = jnp.where(lanes == a0, w0, 0.0) + jnp.where(lanes == a1, w1, 0.0)


def _bookkeep_body(gates_ref, scat_idx_ref, be_ref):
    g = gates_ref[...]
    lanes = lax.broadcasted_iota(jnp.int32, (T, E), 1)
    a0 = jnp.argmax(g, axis=1).reshape(-1, 1)
    oh0 = (lanes == a0).astype(jnp.float32)
    masked = jnp.where(lanes == a0, -jnp.inf, g)
    a1 = jnp.argmax(masked, axis=1).reshape(-1, 1)
    oh1 = (lanes == a1).astype(jnp.float32)

    oh = jnp.concatenate([oh0, oh1], axis=0)
    csum = oh
    s = 1
    while s < T2:
        csum = csum + jnp.concatenate(
            [jnp.zeros((s, E), jnp.float32), csum[:T2 - s]], axis=0)
        s *= 2
    rank = jnp.sum(csum * oh, axis=1, keepdims=True) - 1.0
    count = csum[T2 - 1:T2, :]

    cap = jnp.floor((count + (BLK - 1)) / BLK)
    le = lax.broadcasted_iota(jnp.int32, (E, E), 0).astype(jnp.float32)
    lj = lax.broadcasted_iota(jnp.int32, (E, E), 1).astype(jnp.float32)
    umat = (le <= lj).astype(jnp.float32)
    cumcap = lax.dot_general(cap, umat, (((1,), (0,)), ((), ())),
                             precision=lax.Precision.HIGHEST,
                             preferred_element_type=jnp.float32)
    base = (cumcap - cap) * BLK
    slot_ent = jnp.sum(oh * base, axis=1, keepdims=True) + rank
    scat_idx_ref[...] = slot_ent.astype(jnp.int32)

    b = lax.broadcasted_iota(jnp.int32, (BE_PAD, 1), 0).astype(jnp.float32)
    eb = jnp.sum((cumcap <= b).astype(jnp.float32), axis=1, keepdims=True)
    be_ref[...] = jnp.minimum(eb, float(E - 1)).astype(jnp.int32)


def _sc_mesh():
    return plsc.VectorSubcoreMesh(core_axis_name="c", subcore_axis_name="s",
                                  num_cores=NC, num_subcores=NS)


def _sc_dispatch_body(tokens, scat_idx, vsrc, xs,
                      idxw_v, src_v, rows_v, sem_i, sg0, sg1, ss0, ss1):
    wid = lax.axis_index("s") * NC + lax.axis_index("c")
    base = wid * EPW
    stage = [pltpu.async_copy(scat_idx.at[pl.ds(base + c * CH, CH)],
                              idxw_v.at[c], sem_i) for c in range(NCH)]
    pltpu.sync_copy(vsrc.at[pl.ds(base, EPW)], src_v)
    for cp in stage:
        cp.wait()
    sg = (sg0, sg1)
    ss = (ss0, ss1)
    g = {}
    sc = {}
    g[0] = pltpu.async_copy(tokens.at[src_v.at[pl.ds(0, CH)]],
                            rows_v.at[0], sg[0])
    for c in range(NCH):
        g[c].wait()
        sc[c] = pltpu.async_copy(rows_v.at[c % 2], xs.at[idxw_v.at[c]],
                                 ss[c % 2])
        if c >= 1:
            sc[c - 1].wait()
        if c + 1 < NCH:
            g[c + 1] = pltpu.async_copy(
                tokens.at[src_v.at[pl.ds((c + 1) * CH, CH)]],
                rows_v.at[(c + 1) % 2], sg[(c + 1) % 2])
    sc[NCH - 1].wait()


def _sc_dispatch(tokens, scat_idx, vsrc):
    return pl.kernel(
        _sc_dispatch_body,
        out_type=jax.ShapeDtypeStruct((NPAD, H), jnp.float32),
        mesh=_sc_mesh(),
        scratch_types=[pltpu.VMEM((NCH, CH), jnp.int32),
                       pltpu.VMEM((EPW,), jnp.int32),
                       pltpu.VMEM((2, CH, H), jnp.float32),
                       pltpu.SemaphoreType.DMA,
                       pltpu.SemaphoreType.DMA,
                       pltpu.SemaphoreType.DMA,
                       pltpu.SemaphoreType.DMA,
                       pltpu.SemaphoreType.DMA],
    )(tokens, scat_idx, vsrc)


def _sc_unsort_body(ys, scat_idx, ysab, idx_v, rows_v, sg0, sg1, ss0, ss1):
    wid = lax.axis_index("s") * NC + lax.axis_index("c")
    base = wid * EPW
    pltpu.sync_copy(scat_idx.at[pl.ds(base, EPW)], idx_v)
    sg = (sg0, sg1)
    ss = (ss0, ss1)
    g = {}
    st = {}
    g[0] = pltpu.async_copy(ys.at[idx_v.at[pl.ds(0, CH)]], rows_v.at[0], sg[0])
    for c in range(NCH):
        g[c].wait()
        st[c] = pltpu.async_copy(rows_v.at[c % 2],
                                 ysab.at[pl.ds(base + c * CH, CH)], ss[c % 2])
        if c >= 1:
            st[c - 1].wait()
        if c + 1 < NCH:
            g[c + 1] = pltpu.async_copy(
                ys.at[idx_v.at[pl.ds((c + 1) * CH, CH)]],
                rows_v.at[(c + 1) % 2], sg[(c + 1) % 2])
    st[NCH - 1].wait()


def _sc_unsort(ys, scat_idx):
    return pl.kernel(
        _sc_unsort_body,
        out_type=jax.ShapeDtypeStruct((T2, H), jnp.float32),
        mesh=_sc_mesh(),
        scratch_types=[pltpu.VMEM((EPW,), jnp.int32),
                       pltpu.VMEM((2, CH, H), jnp.float32),
                       pltpu.SemaphoreType.DMA,
                       pltpu.SemaphoreType.DMA,
                       pltpu.SemaphoreType.DMA,
                       pltpu.SemaphoreType.DMA],
    )(ys, scat_idx)


def _gelu_exact(x):
    return 0.5 * x * (1.0 + lax.erf(x * 0.7071067811865476))


def _gmm_body(be_ref, xs_ref, w1_ref, w2_ref, ys_ref):
    xb = xs_ref[...].astype(jnp.bfloat16)
    h = lax.dot_general(xb, w1_ref[0], (((1,), (1,)), ((), ())),
                        preferred_element_type=jnp.float32)
    h = _gelu_exact(h).astype(jnp.bfloat16)
    ys_ref[...] = lax.dot_general(h, w2_ref[0], (((1,), (1,)), ((), ())),
                                  preferred_element_type=jnp.float32)


def _combine_body(ya_ref, yb_ref, gates_ref, out_ref):
    g = gates_ref[...]
    lanes = lax.broadcasted_iota(jnp.int32, g.shape, 1)
    w0 = jnp.max(g, axis=1, keepdims=True)
    a0 = jnp.argmax(g, axis=1).reshape(-1, 1)
    w1 = jnp.max(jnp.where(lanes == a0, -jnp.inf, g), axis=1, keepdims=True)
    out_ref[...] = w0 * ya_ref[...] + w1 * yb_ref[...]


@jax.jit
def kernel(x, Wr, W1, W2):
    bsz, seq, hidden = x.shape
    tokens = x.reshape(T, hidden)

    BT = 512
    gates = pl.pallas_call(
        _router_body,
        grid=(T // BT,),
        in_specs=[pl.BlockSpec((BT, H), lambda t: (t, 0)),
                  pl.BlockSpec((E, H), lambda t: (0, 0))],
        out_specs=pl.BlockSpec((BT, E), lambda t: (t, 0)),
        out_shape=jax.ShapeDtypeStruct((T, E), jnp.float32),
    )(tokens, Wr)

    scat_idx, be = pl.pallas_call(
        _bookkeep_body,
        out_shape=(jax.ShapeDtypeStruct((T2, 1), jnp.int32),
                   jax.ShapeDtypeStruct((BE_PAD, 1), jnp.int32)),
    )(gates)
    scat_idx = scat_idx.reshape(T2)
    be = be.reshape(BE_PAD)[:NBLK]

    tok_iota = jnp.arange(T, dtype=jnp.int32)
    vsrc = jnp.concatenate([tok_iota, tok_iota])

    xs = _sc_dispatch(tokens, scat_idx, vsrc)

    ys = pl.pallas_call(
        _gmm_body,
        grid_spec=pltpu.PrefetchScalarGridSpec(
            num_scalar_prefetch=1,
            grid=(NBLK,),
            in_specs=[pl.BlockSpec((BLK, H), lambda b, be_s: (b, 0)),
                      pl.BlockSpec((1, F, H), lambda b, be_s: (be_s[b], 0, 0)),
                      pl.BlockSpec((1, H, F), lambda b, be_s: (be_s[b], 0, 0))],
            out_specs=pl.BlockSpec((BLK, H), lambda b, be_s: (b, 0)),
        ),
        out_shape=jax.ShapeDtypeStruct((NPAD, H), jnp.float32),
        compiler_params=pltpu.CompilerParams(
            dimension_semantics=("arbitrary",)),
    )(be, xs, W1.astype(jnp.bfloat16), W2.astype(jnp.bfloat16))

    ysab = _sc_unsort(ys, scat_idx)

    out = pl.pallas_call(
        _combine_body,
        grid=(T // BT,),
        in_specs=[pl.BlockSpec((BT, H), lambda t: (t, 0)),
                  pl.BlockSpec((BT, H), lambda t: (t + T // BT, 0)),
                  pl.BlockSpec((BT, E), lambda t: (t, 0))],
        out_specs=pl.BlockSpec((BT, H), lambda t: (t, 0)),
        out_shape=jax.ShapeDtypeStruct((T, H), jnp.float32),
    )(ysab, ysab, gates)

    return out.reshape(bsz, seq, hidden)


# speedup vs baseline: 1.0852x; 1.0852x over previous
import functools

import jax
import jax.numpy as jnp
from jax import lax
from jax.experimental import pallas as pl
from jax.experimental.pallas import tpu as pltpu
from jax.experimental.pallas import tpu_sc as plsc

H = 1024
E = 8
F = 2048
T = 8192
T2 = 2 * T
BLK = 256
NBLK = T2 // BLK + E
NPAD = NBLK * BLK
BE_PAD = 128

NC, NS = 2, 16
NW = NC * NS
EPW = T2 // NW
CH = 32
NCH = EPW // CH


def _router_body(x_ref, wr_ref, gates_ref):
    x = x_ref[...]
    logits = lax.dot_general(x, wr_ref[...], (((1,), (1,)), ((), ())),
                             preferred_element_type=jnp.float32)
    lanes = lax.broadcasted_iota(jnp.int32, logits.shape, 1)
    m0 = jnp.max(logits, axis=1, keepdims=True)
    a0 = jnp.argmax(logits, axis=1).reshape(-1, 1)
    masked = jnp.where(lanes == a0, -jnp.inf, logits)
    m1 = jnp.max(masked, axis=1, keepdims=True)
    a1 = jnp.argmax(masked, axis=1).reshape(-1, 1)
    e1 = jnp.exp(m1 - m0)
    w0 = 1.0 / (1.0 + e1)
    w1 = e1 * w0
    gates_ref[...] = jnp.where(lanes == a0, w0, 0.0) + jnp.where(lanes == a1, w1, 0.0)


def _bookkeep_body(gates_ref, scat_idx_ref, be_ref):
    g = gates_ref[...]
    lanes = lax.broadcasted_iota(jnp.int32, (T, E), 1)
    a0 = jnp.argmax(g, axis=1).reshape(-1, 1)
    oh0 = (lanes == a0).astype(jnp.float32)
    masked = jnp.where(lanes == a0, -jnp.inf, g)
    a1 = jnp.argmax(masked, axis=1).reshape(-1, 1)
    oh1 = (lanes == a1).astype(jnp.float32)

    oh = jnp.concatenate([oh0, oh1], axis=0)
    csum = oh
    s = 1
    while s < T2:
        csum = csum + jnp.concatenate(
            [jnp.zeros((s, E), jnp.float32), csum[:T2 - s]], axis=0)
        s *= 2
    rank = jnp.sum(csum * oh, axis=1, keepdims=True) - 1.0
    count = csum[T2 - 1:T2, :]

    cap = jnp.floor((count + (BLK - 1)) / BLK)
    le = lax.broadcasted_iota(jnp.int32, (E, E), 0).astype(jnp.float32)
    lj = lax.broadcasted_iota(jnp.int32, (E, E), 1).astype(jnp.float32)
    umat = (le <= lj).astype(jnp.float32)
    cumcap = lax.dot_general(cap, umat, (((1,), (0,)), ((), ())),
                             precision=lax.Precision.HIGHEST,
                             preferred_element_type=jnp.float32)
    base = (cumcap - cap) * BLK
    slot_ent = jnp.sum(oh * base, axis=1, keepdims=True) + rank
    scat_idx_ref[...] = slot_ent.astype(jnp.int32)

    b = lax.broadcasted_iota(jnp.int32, (BE_PAD, 1), 0).astype(jnp.float32)
    eb = jnp.sum((cumcap <= b).astype(jnp.float32), axis=1, keepdims=True)
    be_ref[...] = jnp.minimum(eb, float(E - 1)).astype(jnp.int32)


def _sc_mesh():
    return plsc.VectorSubcoreMesh(core_axis_name="c", subcore_axis_name="s",
                                  num_cores=NC, num_subcores=NS)


def _sc_dispatch_body(tokens, scat_idx, vsrc, xs,
                      idxw_v, src_v, rows_v, sem_i, sg0, sg1, ss0, ss1):
    wid = lax.axis_index("s") * NC + lax.axis_index("c")
    base = wid * EPW
    stage = [pltpu.async_copy(scat_idx.at[pl.ds(base + c * CH, CH)],
                              idxw_v.at[c], sem_i) for c in range(NCH)]
    pltpu.sync_copy(vsrc.at[pl.ds(base, EPW)], src_v)
    for cp in stage:
        cp.wait()
    sg = (sg0, sg1)
    ss = (ss0, ss1)
    g = {}
    sc = {}
    g[0] = pltpu.async_copy(tokens.at[src_v.at[pl.ds(0, CH)]],
                            rows_v.at[0], sg[0])
    for c in range(NCH):
        g[c].wait()
        sc[c] = pltpu.async_copy(rows_v.at[c % 2], xs.at[idxw_v.at[c]],
                                 ss[c % 2])
        if c >= 1:
            sc[c - 1].wait()
        if c + 1 < NCH:
            g[c + 1] = pltpu.async_copy(
                tokens.at[src_v.at[pl.ds((c + 1) * CH, CH)]],
                rows_v.at[(c + 1) % 2], sg[(c + 1) % 2])
    sc[NCH - 1].wait()


def _sc_dispatch(tokens, scat_idx, vsrc):
    return pl.kernel(
        _sc_dispatch_body,
        out_type=jax.ShapeDtypeStruct((NPAD, H), jnp.float32),
        mesh=_sc_mesh(),
        scratch_types=[pltpu.VMEM((NCH, CH), jnp.int32),
                       pltpu.VMEM((EPW,), jnp.int32),
                       pltpu.VMEM((2, CH, H), jnp.float32),
                       pltpu.SemaphoreType.DMA,
                       pltpu.SemaphoreType.DMA,
                       pltpu.SemaphoreType.DMA,
                       pltpu.SemaphoreType.DMA,
                       pltpu.SemaphoreType.DMA],
    )(tokens, scat_idx, vsrc)


def _sc_unsort_body(ys, scat_idx, ysab, idx_v, rows_v, sg0, sg1, ss0, ss1):
    wid = lax.axis_index("s") * NC + lax.axis_index("c")
    base = wid * EPW
    pltpu.sync_copy(scat_idx.at[pl.ds(base, EPW)], idx_v)
    sg = (sg0, sg1)
    ss = (ss0, ss1)
    g = {}
    st = {}
    g[0] = pltpu.async_copy(ys.at[idx_v.at[pl.ds(0, CH)]], rows_v.at[0], sg[0])
    for c in range(NCH):
        g[c].wait()
        st[c] = pltpu.async_copy(rows_v.at[c % 2],
                                 ysab.at[pl.ds(base + c * CH, CH)], ss[c % 2])
        if c >= 1:
            st[c - 1].wait()
        if c + 1 < NCH:
            g[c + 1] = pltpu.async_copy(
                ys.at[idx_v.at[pl.ds((c + 1) * CH, CH)]],
                rows_v.at[(c + 1) % 2], sg[(c + 1) % 2])
    st[NCH - 1].wait()


def _sc_unsort(ys, scat_idx):
    return pl.kernel(
        _sc_unsort_body,
        out_type=jax.ShapeDtypeStruct((T2, H), jnp.float32),
        mesh=_sc_mesh(),
        scratch_types=[pltpu.VMEM((EPW,), jnp.int32),
                       pltpu.VMEM((2, CH, H), jnp.float32),
                       pltpu.SemaphoreType.DMA,
                       pltpu.SemaphoreType.DMA,
                       pltpu.SemaphoreType.DMA,
                       pltpu.SemaphoreType.DMA],
    )(ys, scat_idx)


def _gelu_exact(x):
    return 0.5 * x * (1.0 + lax.erf(x * 0.7071067811865476))


def _gmm_body(be_ref, xs_ref, w1_ref, w2_ref, ys_ref):
    h = lax.dot_general(xs_ref[...], w1_ref[0], (((1,), (1,)), ((), ())),
                        preferred_element_type=jnp.float32)
    h = _gelu_exact(h)
    ys_ref[...] = lax.dot_general(h, w2_ref[0], (((1,), (1,)), ((), ())),
                                  preferred_element_type=jnp.float32)


def _combine_body(ya_ref, yb_ref, gates_ref, out_ref):
    g = gates_ref[...]
    lanes = lax.broadcasted_iota(jnp.int32, g.shape, 1)
    w0 = jnp.max(g, axis=1, keepdims=True)
    a0 = jnp.argmax(g, axis=1).reshape(-1, 1)
    w1 = jnp.max(jnp.where(lanes == a0, -jnp.inf, g), axis=1, keepdims=True)
    out_ref[...] = w0 * ya_ref[...] + w1 * yb_ref[...]


@jax.jit
def kernel(x, Wr, W1, W2):
    bsz, seq, hidden = x.shape
    tokens = x.reshape(T, hidden)

    BT = 512
    gates = pl.pallas_call(
        _router_body,
        grid=(T // BT,),
        in_specs=[pl.BlockSpec((BT, H), lambda t: (t, 0)),
                  pl.BlockSpec((E, H), lambda t: (0, 0))],
        out_specs=pl.BlockSpec((BT, E), lambda t: (t, 0)),
        out_shape=jax.ShapeDtypeStruct((T, E), jnp.float32),
    )(tokens, Wr)

    scat_idx, be = pl.pallas_call(
        _bookkeep_body,
        out_shape=(jax.ShapeDtypeStruct((T2, 1), jnp.int32),
                   jax.ShapeDtypeStruct((BE_PAD, 1), jnp.int32)),
    )(gates)
    scat_idx = scat_idx.reshape(T2)
    be = be.reshape(BE_PAD)[:NBLK]

    tok_iota = jnp.arange(T, dtype=jnp.int32)
    vsrc = jnp.concatenate([tok_iota, tok_iota])

    xs = _sc_dispatch(tokens, scat_idx, vsrc)

    ys = pl.pallas_call(
        _gmm_body,
        grid_spec=pltpu.PrefetchScalarGridSpec(
            num_scalar_prefetch=1,
            grid=(NBLK,),
            in_specs=[pl.BlockSpec((BLK, H), lambda b, be_s: (b, 0)),
                      pl.BlockSpec((1, F, H), lambda b, be_s: (be_s[b], 0, 0)),
                      pl.BlockSpec((1, H, F), lambda b, be_s: (be_s[b], 0, 0))],
            out_specs=pl.BlockSpec((BLK, H), lambda b, be_s: (b, 0)),
        ),
        out_shape=jax.ShapeDtypeStruct((NPAD, H), jnp.float32),
        compiler_params=pltpu.CompilerParams(
            dimension_semantics=("parallel",)),
    )(be, xs, W1, W2)

    ysab = _sc_unsort(ys, scat_idx)

    out = pl.pallas_call(
        _combine_body,
        grid=(T // BT,),
        in_specs=[pl.BlockSpec((BT, H), lambda t: (t, 0)),
                  pl.BlockSpec((BT, H), lambda t: (t + T // BT, 0)),
                  pl.BlockSpec((BT, E), lambda t: (t, 0))],
        out_specs=pl.BlockSpec((BT, H), lambda t: (t, 0)),
        out_shape=jax.ShapeDtypeStruct((T, H), jnp.float32),
    )(ysab, ysab, gates)

    return out.reshape(bsz, seq, hidden)


# speedup vs baseline: 1.1134x; 1.0260x over previous
import functools

import jax
import jax.numpy as jnp
from jax import lax
from jax.experimental import pallas as pl
from jax.experimental.pallas import tpu as pltpu
from jax.experimental.pallas import tpu_sc as plsc

H = 1024
E = 8
F = 2048
T = 8192
T2 = 2 * T
BLK = 512
NBLK = T2 // BLK + E
NPAD = NBLK * BLK
BE_PAD = 128

NC, NS = 2, 16
NW = NC * NS
EPW = T2 // NW
CH = 32
NCH = EPW // CH


def _router_body(x_ref, wr_ref, gates_ref):
    x = x_ref[...]
    logits = lax.dot_general(x, wr_ref[...], (((1,), (1,)), ((), ())),
                             preferred_element_type=jnp.float32)
    lanes = lax.broadcasted_iota(jnp.int32, logits.shape, 1)
    m0 = jnp.max(logits, axis=1, keepdims=True)
    a0 = jnp.argmax(logits, axis=1).reshape(-1, 1)
    masked = jnp.where(lanes == a0, -jnp.inf, logits)
    m1 = jnp.max(masked, axis=1, keepdims=True)
    a1 = jnp.argmax(masked, axis=1).reshape(-1, 1)
    e1 = jnp.exp(m1 - m0)
    w0 = 1.0 / (1.0 + e1)
    w1 = e1 * w0
    gates_ref[...] = jnp.where(lanes == a0, w0, 0.0) + jnp.where(lanes == a1, w1, 0.0)


def _bookkeep_body(gates_ref, scat_idx_ref, be_ref):
    g = gates_ref[...]
    lanes = lax.broadcasted_iota(jnp.int32, (T, E), 1)
    a0 = jnp.argmax(g, axis=1).reshape(-1, 1)
    oh0 = (lanes == a0).astype(jnp.float32)
    masked = jnp.where(lanes == a0, -jnp.inf, g)
    a1 = jnp.argmax(masked, axis=1).reshape(-1, 1)
    oh1 = (lanes == a1).astype(jnp.float32)

    oh = jnp.concatenate([oh0, oh1], axis=0)
    csum = oh
    s = 1
    while s < T2:
        csum = csum + jnp.concatenate(
            [jnp.zeros((s, E), jnp.float32), csum[:T2 - s]], axis=0)
        s *= 2
    rank = jnp.sum(csum * oh, axis=1, keepdims=True) - 1.0
    count = csum[T2 - 1:T2, :]

    cap = jnp.floor((count + (BLK - 1)) / BLK)
    le = lax.broadcasted_iota(jnp.int32, (E, E), 0).astype(jnp.float32)
    lj = lax.broadcasted_iota(jnp.int32, (E, E), 1).astype(jnp.float32)
    umat = (le <= lj).astype(jnp.float32)
    cumcap = lax.dot_general(cap, umat, (((1,), (0,)), ((), ())),
                             precision=lax.Precision.HIGHEST,
                             preferred_element_type=jnp.float32)
    base = (cumcap - cap) * BLK
    slot_ent = jnp.sum(oh * base, axis=1, keepdims=True) + rank
    scat_idx_ref[...] = slot_ent.astype(jnp.int32)

    b = lax.broadcasted_iota(jnp.int32, (BE_PAD, 1), 0).astype(jnp.float32)
    eb = jnp.sum((cumcap <= b).astype(jnp.float32), axis=1, keepdims=True)
    be_ref[...] = jnp.minimum(eb, float(E - 1)).astype(jnp.int32)


def _sc_mesh():
    return plsc.VectorSubcoreMesh(core_axis_name="c", subcore_axis_name="s",
                                  num_cores=NC, num_subcores=NS)


def _sc_dispatch_body(tokens, scat_idx, vsrc, xs,
                      idxw_v, src_v, rows_v, sem_i, sg0, sg1, ss0, ss1):
    wid = lax.axis_index("s") * NC + lax.axis_index("c")
    base = wid * EPW
    stage = [pltpu.async_copy(scat_idx.at[pl.ds(base + c * CH, CH)],
                              idxw_v.at[c], sem_i) for c in range(NCH)]
    pltpu.sync_copy(vsrc.at[pl.ds(base, EPW)], src_v)
    for cp in stage:
        cp.wait()
    sg = (sg0, sg1)
    ss = (ss0, ss1)
    g = {}
    sc = {}
    g[0] = pltpu.async_copy(tokens.at[src_v.at[pl.ds(0, CH)]],
                            rows_v.at[0], sg[0])
    for c in range(NCH):
        g[c].wait()
        sc[c] = pltpu.async_copy(rows_v.at[c % 2], xs.at[idxw_v.at[c]],
                                 ss[c % 2])
        if c >= 1:
            sc[c - 1].wait()
        if c + 1 < NCH:
            g[c + 1] = pltpu.async_copy(
                tokens.at[src_v.at[pl.ds((c + 1) * CH, CH)]],
                rows_v.at[(c + 1) % 2], sg[(c + 1) % 2])
    sc[NCH - 1].wait()


def _sc_dispatch(tokens, scat_idx, vsrc):
    return pl.kernel(
        _sc_dispatch_body,
        out_type=jax.ShapeDtypeStruct((NPAD, H), jnp.float32),
        mesh=_sc_mesh(),
        scratch_types=[pltpu.VMEM((NCH, CH), jnp.int32),
                       pltpu.VMEM((EPW,), jnp.int32),
                       pltpu.VMEM((2, CH, H), jnp.float32),
                       pltpu.SemaphoreType.DMA,
                       pltpu.SemaphoreType.DMA,
                       pltpu.SemaphoreType.DMA,
                       pltpu.SemaphoreType.DMA,
                       pltpu.SemaphoreType.DMA],
    )(tokens, scat_idx, vsrc)


def _sc_unsort_body(ys, scat_idx, ysab, idx_v, rows_v, sg0, sg1, ss0, ss1):
    wid = lax.axis_index("s") * NC + lax.axis_index("c")
    base = wid * EPW
    pltpu.sync_copy(scat_idx.at[pl.ds(base, EPW)], idx_v)
    sg = (sg0, sg1)
    ss = (ss0, ss1)
    g = {}
    st = {}
    g[0] = pltpu.async_copy(ys.at[idx_v.at[pl.ds(0, CH)]], rows_v.at[0], sg[0])
    for c in range(NCH):
        g[c].wait()
        st[c] = pltpu.async_copy(rows_v.at[c % 2],
                                 ysab.at[pl.ds(base + c * CH, CH)], ss[c % 2])
        if c >= 1:
            st[c - 1].wait()
        if c + 1 < NCH:
            g[c + 1] = pltpu.async_copy(
                ys.at[idx_v.at[pl.ds((c + 1) * CH, CH)]],
                rows_v.at[(c + 1) % 2], sg[(c + 1) % 2])
    st[NCH - 1].wait()


def _sc_unsort(ys, scat_idx):
    return pl.kernel(
        _sc_unsort_body,
        out_type=jax.ShapeDtypeStruct((T2, H), jnp.float32),
        mesh=_sc_mesh(),
        scratch_types=[pltpu.VMEM((EPW,), jnp.int32),
                       pltpu.VMEM((2, CH, H), jnp.float32),
                       pltpu.SemaphoreType.DMA,
                       pltpu.SemaphoreType.DMA,
                       pltpu.SemaphoreType.DMA,
                       pltpu.SemaphoreType.DMA],
    )(ys, scat_idx)


def _gelu_exact(x):
    return 0.5 * x * (1.0 + lax.erf(x * 0.7071067811865476))


def _gmm_body(be_ref, xs_ref, w1_ref, w2_ref, ys_ref):
    h = lax.dot_general(xs_ref[...], w1_ref[0], (((1,), (1,)), ((), ())),
                        preferred_element_type=jnp.float32)
    h = _gelu_exact(h)
    ys_ref[...] = lax.dot_general(h, w2_ref[0], (((1,), (1,)), ((), ())),
                                  preferred_element_type=jnp.float32)


def _combine_body(ya_ref, yb_ref, gates_ref, out_ref):
    g = gates_ref[...]
    lanes = lax.broadcasted_iota(jnp.int32, g.shape, 1)
    w0 = jnp.max(g, axis=1, keepdims=True)
    a0 = jnp.argmax(g, axis=1).reshape(-1, 1)
    w1 = jnp.max(jnp.where(lanes == a0, -jnp.inf, g), axis=1, keepdims=True)
    out_ref[...] = w0 * ya_ref[...] + w1 * yb_ref[...]


@jax.jit
def kernel(x, Wr, W1, W2):
    bsz, seq, hidden = x.shape
    tokens = x.reshape(T, hidden)

    BT = 512
    gates = pl.pallas_call(
        _router_body,
        grid=(T // BT,),
        in_specs=[pl.BlockSpec((BT, H), lambda t: (t, 0)),
                  pl.BlockSpec((E, H), lambda t: (0, 0))],
        out_specs=pl.BlockSpec((BT, E), lambda t: (t, 0)),
        out_shape=jax.ShapeDtypeStruct((T, E), jnp.float32),
    )(tokens, Wr)

    scat_idx, be = pl.pallas_call(
        _bookkeep_body,
        out_shape=(jax.ShapeDtypeStruct((T2, 1), jnp.int32),
                   jax.ShapeDtypeStruct((BE_PAD, 1), jnp.int32)),
    )(gates)
    scat_idx = scat_idx.reshape(T2)
    be = be.reshape(BE_PAD)[:NBLK]

    tok_iota = jnp.arange(T, dtype=jnp.int32)
    vsrc = jnp.concatenate([tok_iota, tok_iota])

    xs = _sc_dispatch(tokens, scat_idx, vsrc)

    ys = pl.pallas_call(
        _gmm_body,
        grid_spec=pltpu.PrefetchScalarGridSpec(
            num_scalar_prefetch=1,
            grid=(NBLK,),
            in_specs=[pl.BlockSpec((BLK, H), lambda b, be_s: (b, 0)),
                      pl.BlockSpec((1, F, H), lambda b, be_s: (be_s[b], 0, 0)),
                      pl.BlockSpec((1, H, F), lambda b, be_s: (be_s[b], 0, 0))],
            out_specs=pl.BlockSpec((BLK, H), lambda b, be_s: (b, 0)),
        ),
        out_shape=jax.ShapeDtypeStruct((NPAD, H), jnp.float32),
        compiler_params=pltpu.CompilerParams(
            dimension_semantics=("parallel",)),
    )(be, xs, W1, W2)

    ysab = _sc_unsort(ys, scat_idx)

    out = pl.pallas_call(
        _combine_body,
        grid=(T // BT,),
        in_specs=[pl.BlockSpec((BT, H), lambda t: (t, 0)),
                  pl.BlockSpec((BT, H), lambda t: (t + T // BT, 0)),
                  pl.BlockSpec((BT, E), lambda t: (t, 0))],
        out_specs=pl.BlockSpec((BT, H), lambda t: (t, 0)),
        out_shape=jax.ShapeDtypeStruct((T, H), jnp.float32),
    )(ysab, ysab, gates)

    return out.reshape(bsz, seq, hidden)


# speedup vs baseline: 1.1304x; 1.0153x over previous
import functools

import jax
import jax.numpy as jnp
from jax import lax
from jax.experimental import pallas as pl
from jax.experimental.pallas import tpu as pltpu
from jax.experimental.pallas import tpu_sc as plsc

H = 1024
E = 8
F = 2048
T = 8192
T2 = 2 * T
BLK = 512
NBLK = T2 // BLK + E
NPAD = NBLK * BLK
BE_PAD = 128

NC, NS = 2, 16
NW = NC * NS
EPW = T2 // NW
CH = 32
NCH = EPW // CH


def _router_body(x_ref, wr_ref, gates_ref):
    x = x_ref[...]
    logits = lax.dot_general(x, wr_ref[...], (((1,), (1,)), ((), ())),
                             preferred_element_type=jnp.float32)
    lanes = lax.broadcasted_iota(jnp.int32, logits.shape, 1)
    m0 = jnp.max(logits, axis=1, keepdims=True)
    a0 = jnp.argmax(logits, axis=1).reshape(-1, 1)
    masked = jnp.where(lanes == a0, -jnp.inf, logits)
    m1 = jnp.max(masked, axis=1, keepdims=True)
    a1 = jnp.argmax(masked, axis=1).reshape(-1, 1)
    e1 = jnp.exp(m1 - m0)
    w0 = 1.0 / (1.0 + e1)
    w1 = e1 * w0
    gates_ref[...] = jnp.where(lanes == a0, w0, 0.0) + jnp.where(lanes == a1, w1, 0.0)


def _bookkeep_body(gates_ref, scat_idx_ref, be_ref):
    g = gates_ref[...]
    lanes = lax.broadcasted_iota(jnp.int32, (T, E), 1)
    a0 = jnp.argmax(g, axis=1).reshape(-1, 1)
    oh0 = (lanes == a0).astype(jnp.float32)
    masked = jnp.where(lanes == a0, -jnp.inf, g)
    a1 = jnp.argmax(masked, axis=1).reshape(-1, 1)
    oh1 = (lanes == a1).astype(jnp.float32)

    oh = jnp.concatenate([oh0, oh1], axis=0)
    csum = oh
    s = 1
    while s < T2:
        csum = csum + jnp.concatenate(
            [jnp.zeros((s, E), jnp.float32), csum[:T2 - s]], axis=0)
        s *= 2
    rank = jnp.sum(csum * oh, axis=1, keepdims=True) - 1.0
    count = csum[T2 - 1:T2, :]

    cap = jnp.floor((count + (BLK - 1)) / BLK)
    le = lax.broadcasted_iota(jnp.int32, (E, E), 0).astype(jnp.float32)
    lj = lax.broadcasted_iota(jnp.int32, (E, E), 1).astype(jnp.float32)
    umat = (le <= lj).astype(jnp.float32)
    cumcap = lax.dot_general(cap, umat, (((1,), (0,)), ((), ())),
                             precision=lax.Precision.HIGHEST,
                             preferred_element_type=jnp.float32)
    base = (cumcap - cap) * BLK
    slot_ent = jnp.sum(oh * base, axis=1, keepdims=True) + rank
    scat_idx_ref[...] = slot_ent.astype(jnp.int32)

    b = lax.broadcasted_iota(jnp.int32, (BE_PAD, 1), 0).astype(jnp.float32)
    eb = jnp.sum((cumcap <= b).astype(jnp.float32), axis=1, keepdims=True)
    be_ref[...] = jnp.minimum(eb, float(E - 1)).astype(jnp.int32)


def _sc_mesh():
    return plsc.VectorSubcoreMesh(core_axis_name="c", subcore_axis_name="s",
                                  num_cores=NC, num_subcores=NS)


def _sc_dispatch_body(tokens, scat_idx, vsrc, xs,
                      idxw_v, src_v, rows_v, sem_i,
                      sg0, sg1, sg2, ss0, ss1, ss2):
    wid = lax.axis_index("s") * NC + lax.axis_index("c")
    base = wid * EPW
    stage = [pltpu.async_copy(scat_idx.at[pl.ds(base + c * CH, CH)],
                              idxw_v.at[c], sem_i) for c in range(NCH)]
    pltpu.sync_copy(vsrc.at[pl.ds(base, EPW)], src_v)
    for cp in stage:
        cp.wait()
    sg = (sg0, sg1, sg2)
    ss = (ss0, ss1, ss2)
    g = {}
    sc = {}
    g[0] = pltpu.async_copy(tokens.at[src_v.at[pl.ds(0, CH)]],
                            rows_v.at[0], sg[0])
    g[1] = pltpu.async_copy(tokens.at[src_v.at[pl.ds(CH, CH)]],
                            rows_v.at[1], sg[1])
    for c in range(NCH):
        g[c].wait()
        sc[c] = pltpu.async_copy(rows_v.at[c % 3], xs.at[idxw_v.at[c]],
                                 ss[c % 3])
        if c >= 1:
            sc[c - 1].wait()
        if c + 2 < NCH:
            g[c + 2] = pltpu.async_copy(
                tokens.at[src_v.at[pl.ds((c + 2) * CH, CH)]],
                rows_v.at[(c + 2) % 3], sg[(c + 2) % 3])
    sc[NCH - 1].wait()


def _sc_dispatch(tokens, scat_idx, vsrc):
    return pl.kernel(
        _sc_dispatch_body,
        out_type=jax.ShapeDtypeStruct((NPAD, H), jnp.float32),
        mesh=_sc_mesh(),
        scratch_types=[pltpu.VMEM((NCH, CH), jnp.int32),
                       pltpu.VMEM((EPW,), jnp.int32),
                       pltpu.VMEM((3, CH, H), jnp.float32),
                       pltpu.SemaphoreType.DMA,
                       pltpu.SemaphoreType.DMA,
                       pltpu.SemaphoreType.DMA,
                       pltpu.SemaphoreType.DMA,
                       pltpu.SemaphoreType.DMA,
                       pltpu.SemaphoreType.DMA,
                       pltpu.SemaphoreType.DMA],
    )(tokens, scat_idx, vsrc)


def _sc_unsort_body(ys, scat_idx, ysab, idx_v, rows_v,
                    sg0, sg1, sg2, ss0, ss1, ss2):
    wid = lax.axis_index("s") * NC + lax.axis_index("c")
    base = wid * EPW
    pltpu.sync_copy(scat_idx.at[pl.ds(base, EPW)], idx_v)
    sg = (sg0, sg1, sg2)
    ss = (ss0, ss1, ss2)
    g = {}
    st = {}
    g[0] = pltpu.async_copy(ys.at[idx_v.at[pl.ds(0, CH)]], rows_v.at[0], sg[0])
    g[1] = pltpu.async_copy(ys.at[idx_v.at[pl.ds(CH, CH)]], rows_v.at[1], sg[1])
    for c in range(NCH):
        g[c].wait()
        st[c] = pltpu.async_copy(rows_v.at[c % 3],
                                 ysab.at[pl.ds(base + c * CH, CH)], ss[c % 3])
        if c >= 1:
            st[c - 1].wait()
        if c + 2 < NCH:
            g[c + 2] = pltpu.async_copy(
                ys.at[idx_v.at[pl.ds((c + 2) * CH, CH)]],
                rows_v.at[(c + 2) % 3], sg[(c + 2) % 3])
    st[NCH - 1].wait()


def _sc_unsort(ys, scat_idx):
    return pl.kernel(
        _sc_unsort_body,
        out_type=jax.ShapeDtypeStruct((T2, H), jnp.float32),
        mesh=_sc_mesh(),
        scratch_types=[pltpu.VMEM((EPW,), jnp.int32),
                       pltpu.VMEM((3, CH, H), jnp.float32),
                       pltpu.SemaphoreType.DMA,
                       pltpu.SemaphoreType.DMA,
                       pltpu.SemaphoreType.DMA,
                       pltpu.SemaphoreType.DMA,
                       pltpu.SemaphoreType.DMA,
                       pltpu.SemaphoreType.DMA],
    )(ys, scat_idx)


def _gelu_exact(x):
    return 0.5 * x * (1.0 + lax.erf(x * 0.7071067811865476))


def _gmm_body(be_ref, xs_ref, w1_ref, w2_ref, ys_ref):
    h = lax.dot_general(xs_ref[...], w1_ref[0], (((1,), (1,)), ((), ())),
                        preferred_element_type=jnp.float32)
    h = _gelu_exact(h)
    ys_ref[...] = lax.dot_general(h, w2_ref[0], (((1,), (1,)), ((), ())),
                                  preferred_element_type=jnp.float32)


def _combine_body(ya_ref, yb_ref, gates_ref, out_ref):
    g = gates_ref[...]
    lanes = lax.broadcasted_iota(jnp.int32, g.shape, 1)
    w0 = jnp.max(g, axis=1, keepdims=True)
    a0 = jnp.argmax(g, axis=1).reshape(-1, 1)
    w1 = jnp.max(jnp.where(lanes == a0, -jnp.inf, g), axis=1, keepdims=True)
    out_ref[...] = w0 * ya_ref[...] + w1 * yb_ref[...]


@jax.jit
def kernel(x, Wr, W1, W2):
    bsz, seq, hidden = x.shape
    tokens = x.reshape(T, hidden)

    BT = 512
    gates = pl.pallas_call(
        _router_body,
        grid=(T // BT,),
        in_specs=[pl.BlockSpec((BT, H), lambda t: (t, 0)),
                  pl.BlockSpec((E, H), lambda t: (0, 0))],
        out_specs=pl.BlockSpec((BT, E), lambda t: (t, 0)),
        out_shape=jax.ShapeDtypeStruct((T, E), jnp.float32),
    )(tokens, Wr)

    scat_idx, be = pl.pallas_call(
        _bookkeep_body,
        out_shape=(jax.ShapeDtypeStruct((T2, 1), jnp.int32),
                   jax.ShapeDtypeStruct((BE_PAD, 1), jnp.int32)),
    )(gates)
    scat_idx = scat_idx.reshape(T2)
    be = be.reshape(BE_PAD)[:NBLK]

    tok_iota = jnp.arange(T, dtype=jnp.int32)
    vsrc = jnp.concatenate([tok_iota, tok_iota])

    xs = _sc_dispatch(tokens, scat_idx, vsrc)

    ys = pl.pallas_call(
        _gmm_body,
        grid_spec=pltpu.PrefetchScalarGridSpec(
            num_scalar_prefetch=1,
            grid=(NBLK,),
            in_specs=[pl.BlockSpec((BLK, H), lambda b, be_s: (b, 0)),
                      pl.BlockSpec((1, F, H), lambda b, be_s: (be_s[b], 0, 0)),
                      pl.BlockSpec((1, H, F), lambda b, be_s: (be_s[b], 0, 0))],
            out_specs=pl.BlockSpec((BLK, H), lambda b, be_s: (b, 0)),
        ),
        out_shape=jax.ShapeDtypeStruct((NPAD, H), jnp.float32),
        compiler_params=pltpu.CompilerParams(
            dimension_semantics=("parallel",)),
    )(be, xs, W1, W2)

    ysab = _sc_unsort(ys, scat_idx)

    out = pl.pallas_call(
        _combine_body,
        grid=(T // BT,),
        in_specs=[pl.BlockSpec((BT, H), lambda t: (t, 0)),
                  pl.BlockSpec((BT, H), lambda t: (t + T // BT, 0)),
                  pl.BlockSpec((BT, E), lambda t: (t, 0))],
        out_specs=pl.BlockSpec((BT, H), lambda t: (t, 0)),
        out_shape=jax.ShapeDtypeStruct((T, H), jnp.float32),
    )(ysab, ysab, gates)

    return out.reshape(bsz, seq, hidden)


# speedup vs baseline: 1.1430x; 1.0111x over previous
import functools

import jax
import jax.numpy as jnp
from jax import lax
from jax.experimental import pallas as pl
from jax.experimental.pallas import tpu as pltpu
from jax.experimental.pallas import tpu_sc as plsc

H = 1024
E = 8
F = 2048
T = 8192
T2 = 2 * T
BLK = 512
NBLK = T2 // BLK + E
NPAD = NBLK * BLK
BE_PAD = 128

NC, NS = 2, 16
NW = NC * NS
EPW = T2 // NW
CH = 32
NCH = EPW // CH


def _router_bookkeep_body(x_ref, wr_ref, gates_ref, scat_idx_ref, be_ref,
                          gsc_ref):
    t = pl.program_id(0)

    @pl.when(t < T // 512)
    def _():
        _router_step(x_ref, wr_ref, gates_ref, gsc_ref, t)

    @pl.when(t == T // 512)
    def _():
        _bookkeep_step(gsc_ref, scat_idx_ref, be_ref)


def _router_step(x_ref, wr_ref, gates_ref, gsc_ref, t):
    x = x_ref[...]
    logits = lax.dot_general(x, wr_ref[...], (((1,), (1,)), ((), ())),
                             preferred_element_type=jnp.float32)
    lanes = lax.broadcasted_iota(jnp.int32, logits.shape, 1)
    m0 = jnp.max(logits, axis=1, keepdims=True)
    a0 = jnp.argmax(logits, axis=1).reshape(-1, 1)
    masked = jnp.where(lanes == a0, -jnp.inf, logits)
    m1 = jnp.max(masked, axis=1, keepdims=True)
    a1 = jnp.argmax(masked, axis=1).reshape(-1, 1)
    e1 = jnp.exp(m1 - m0)
    w0 = 1.0 / (1.0 + e1)
    w1 = e1 * w0
    gblk = jnp.where(lanes == a0, w0, 0.0) + jnp.where(lanes == a1, w1, 0.0)
    gates_ref[...] = gblk
    gsc_ref[pl.ds(t * 512, 512), :] = gblk


def _bookkeep_step(gsc_ref, scat_idx_ref, be_ref):
    g = gsc_ref[...]
    lanes = lax.broadcasted_iota(jnp.int32, (T, E), 1)
    a0 = jnp.argmax(g, axis=1).reshape(-1, 1)
    oh0 = (lanes == a0).astype(jnp.float32)
    masked = jnp.where(lanes == a0, -jnp.inf, g)
    a1 = jnp.argmax(masked, axis=1).reshape(-1, 1)
    oh1 = (lanes == a1).astype(jnp.float32)

    oh = jnp.concatenate([oh0, oh1], axis=0)
    csum = oh
    s = 1
    while s < T2:
        csum = csum + jnp.concatenate(
            [jnp.zeros((s, E), jnp.float32), csum[:T2 - s]], axis=0)
        s *= 2
    rank = jnp.sum(csum * oh, axis=1, keepdims=True) - 1.0
    count = csum[T2 - 1:T2, :]

    cap = jnp.floor((count + (BLK - 1)) / BLK)
    le = lax.broadcasted_iota(jnp.int32, (E, E), 0).astype(jnp.float32)
    lj = lax.broadcasted_iota(jnp.int32, (E, E), 1).astype(jnp.float32)
    umat = (le <= lj).astype(jnp.float32)
    cumcap = lax.dot_general(cap, umat, (((1,), (0,)), ((), ())),
                             precision=lax.Precision.HIGHEST,
                             preferred_element_type=jnp.float32)
    base = (cumcap - cap) * BLK
    slot_ent = jnp.sum(oh * base, axis=1, keepdims=True) + rank
    scat_idx_ref[...] = slot_ent.astype(jnp.int32)

    b = lax.broadcasted_iota(jnp.int32, (BE_PAD, 1), 0).astype(jnp.float32)
    eb = jnp.sum((cumcap <= b).astype(jnp.float32), axis=1, keepdims=True)
    be_ref[...] = jnp.minimum(eb, float(E - 1)).astype(jnp.int32)


def _sc_mesh():
    return plsc.VectorSubcoreMesh(core_axis_name="c", subcore_axis_name="s",
                                  num_cores=NC, num_subcores=NS)


def _sc_dispatch_body(tokens, scat_idx, vsrc, xs,
                      idxw_v, src_v, rows_v, sem_i,
                      sg0, sg1, sg2, ss0, ss1, ss2):
    wid = lax.axis_index("s") * NC + lax.axis_index("c")
    base = wid * EPW
    stage = [pltpu.async_copy(scat_idx.at[pl.ds(base + c * CH, CH)],
                              idxw_v.at[c], sem_i) for c in range(NCH)]
    pltpu.sync_copy(vsrc.at[pl.ds(base, EPW)], src_v)
    for cp in stage:
        cp.wait()
    sg = (sg0, sg1, sg2)
    ss = (ss0, ss1, ss2)
    g = {}
    sc = {}
    g[0] = pltpu.async_copy(tokens.at[src_v.at[pl.ds(0, CH)]],
                            rows_v.at[0], sg[0])
    g[1] = pltpu.async_copy(tokens.at[src_v.at[pl.ds(CH, CH)]],
                            rows_v.at[1], sg[1])
    for c in range(NCH):
        g[c].wait()
        sc[c] = pltpu.async_copy(rows_v.at[c % 3], xs.at[idxw_v.at[c]],
                                 ss[c % 3])
        if c >= 1:
            sc[c - 1].wait()
        if c + 2 < NCH:
            g[c + 2] = pltpu.async_copy(
                tokens.at[src_v.at[pl.ds((c + 2) * CH, CH)]],
                rows_v.at[(c + 2) % 3], sg[(c + 2) % 3])
    sc[NCH - 1].wait()


def _sc_dispatch(tokens, scat_idx, vsrc):
    return pl.kernel(
        _sc_dispatch_body,
        out_type=jax.ShapeDtypeStruct((NPAD, H), jnp.float32),
        mesh=_sc_mesh(),
        scratch_types=[pltpu.VMEM((NCH, CH), jnp.int32),
                       pltpu.VMEM((EPW,), jnp.int32),
                       pltpu.VMEM((3, CH, H), jnp.float32),
                       pltpu.SemaphoreType.DMA,
                       pltpu.SemaphoreType.DMA,
                       pltpu.SemaphoreType.DMA,
                       pltpu.SemaphoreType.DMA,
                       pltpu.SemaphoreType.DMA,
                       pltpu.SemaphoreType.DMA,
                       pltpu.SemaphoreType.DMA],
    )(tokens, scat_idx, vsrc)


def _sc_unsort_body(ys, scat_idx, ysab, idx_v, rows_v,
                    sg0, sg1, sg2, ss0, ss1, ss2):
    wid = lax.axis_index("s") * NC + lax.axis_index("c")
    base = wid * EPW
    pltpu.sync_copy(scat_idx.at[pl.ds(base, EPW)], idx_v)
    sg = (sg0, sg1, sg2)
    ss = (ss0, ss1, ss2)
    g = {}
    st = {}
    g[0] = pltpu.async_copy(ys.at[idx_v.at[pl.ds(0, CH)]], rows_v.at[0], sg[0])
    g[1] = pltpu.async_copy(ys.at[idx_v.at[pl.ds(CH, CH)]], rows_v.at[1], sg[1])
    for c in range(NCH):
        g[c].wait()
        st[c] = pltpu.async_copy(rows_v.at[c % 3],
                                 ysab.at[pl.ds(base + c * CH, CH)], ss[c % 3])
        if c >= 1:
            st[c - 1].wait()
        if c + 2 < NCH:
            g[c + 2] = pltpu.async_copy(
                ys.at[idx_v.at[pl.ds((c + 2) * CH, CH)]],
                rows_v.at[(c + 2) % 3], sg[(c + 2) % 3])
    st[NCH - 1].wait()


def _sc_unsort(ys, scat_idx):
    return pl.kernel(
        _sc_unsort_body,
        out_type=jax.ShapeDtypeStruct((T2, H), jnp.float32),
        mesh=_sc_mesh(),
        scratch_types=[pltpu.VMEM((EPW,), jnp.int32),
                       pltpu.VMEM((3, CH, H), jnp.float32),
                       pltpu.SemaphoreType.DMA,
                       pltpu.SemaphoreType.DMA,
                       pltpu.SemaphoreType.DMA,
                       pltpu.SemaphoreType.DMA,
                       pltpu.SemaphoreType.DMA,
                       pltpu.SemaphoreType.DMA],
    )(ys, scat_idx)


def _gelu_exact(x):
    return 0.5 * x * (1.0 + lax.erf(x * 0.7071067811865476))


def _gmm_body(be_ref, xs_ref, w1_ref, w2_ref, ys_ref):
    h = lax.dot_general(xs_ref[...], w1_ref[0], (((1,), (1,)), ((), ())),
                        preferred_element_type=jnp.float32)
    h = _gelu_exact(h)
    ys_ref[...] = lax.dot_general(h, w2_ref[0], (((1,), (1,)), ((), ())),
                                  preferred_element_type=jnp.float32)


def _combine_body(ya_ref, yb_ref, gates_ref, out_ref):
    g = gates_ref[...]
    lanes = lax.broadcasted_iota(jnp.int32, g.shape, 1)
    w0 = jnp.max(g, axis=1, keepdims=True)
    a0 = jnp.argmax(g, axis=1).reshape(-1, 1)
    w1 = jnp.max(jnp.where(lanes == a0, -jnp.inf, g), axis=1, keepdims=True)
    out_ref[...] = w0 * ya_ref[...] + w1 * yb_ref[...]


@jax.jit
def kernel(x, Wr, W1, W2):
    bsz, seq, hidden = x.shape
    tokens = x.reshape(T, hidden)

    BT = 512
    nrb = T // BT
    gates, scat_idx, be = pl.pallas_call(
        _router_bookkeep_body,
        grid=(nrb + 1,),
        in_specs=[pl.BlockSpec((BT, H), lambda t: (jnp.minimum(t, nrb - 1), 0)),
                  pl.BlockSpec((E, H), lambda t: (0, 0))],
        out_specs=(pl.BlockSpec((BT, E), lambda t: (jnp.minimum(t, nrb - 1), 0)),
                   pl.BlockSpec((T2, 1), lambda t: (0, 0)),
                   pl.BlockSpec((BE_PAD, 1), lambda t: (0, 0))),
        out_shape=(jax.ShapeDtypeStruct((T, E), jnp.float32),
                   jax.ShapeDtypeStruct((T2, 1), jnp.int32),
                   jax.ShapeDtypeStruct((BE_PAD, 1), jnp.int32)),
        scratch_shapes=[pltpu.VMEM((T, E), jnp.float32)],
    )(tokens, Wr)
    scat_idx = scat_idx.reshape(T2)
    be = be.reshape(BE_PAD)[:NBLK]

    tok_iota = jnp.arange(T, dtype=jnp.int32)
    vsrc = jnp.concatenate([tok_iota, tok_iota])

    xs = _sc_dispatch(tokens, scat_idx, vsrc)

    ys = pl.pallas_call(
        _gmm_body,
        grid_spec=pltpu.PrefetchScalarGridSpec(
            num_scalar_prefetch=1,
            grid=(NBLK,),
            in_specs=[pl.BlockSpec((BLK, H), lambda b, be_s: (b, 0)),
                      pl.BlockSpec((1, F, H), lambda b, be_s: (be_s[b], 0, 0)),
                      pl.BlockSpec((1, H, F), lambda b, be_s: (be_s[b], 0, 0))],
            out_specs=pl.BlockSpec((BLK, H), lambda b, be_s: (b, 0)),
        ),
        out_shape=jax.ShapeDtypeStruct((NPAD, H), jnp.float32),
        compiler_params=pltpu.CompilerParams(
            dimension_semantics=("parallel",)),
    )(be, xs, W1, W2)

    ysab = _sc_unsort(ys, scat_idx)

    out = pl.pallas_call(
        _combine_body,
        grid=(T // BT,),
        in_specs=[pl.BlockSpec((BT, H), lambda t: (t, 0)),
                  pl.BlockSpec((BT, H), lambda t: (t + T // BT, 0)),
                  pl.BlockSpec((BT, E), lambda t: (t, 0))],
        out_specs=pl.BlockSpec((BT, H), lambda t: (t, 0)),
        out_shape=jax.ShapeDtypeStruct((T, H), jnp.float32),
    )(ysab, ysab, gates)

    return out.reshape(bsz, seq, hidden)
